# baseline (speedup 1.0000x reference)
import functools

import jax
import jax.numpy as jnp
from jax import lax
from jax.experimental import pallas as pl
from jax.experimental.pallas import tpu as pltpu
from jax.experimental.pallas import tpu_sc as plsc

B, N = 128, 8192
L = 16
NC, NS = 2, 16
NW = NC * NS
SC_ROWS = 32
TC_ROWS = B - SC_ROWS
NWIN = 2
CW = N // NWIN
WCHUNKS = CW // L

CB = 2048
NBLK = N // CB
G = 256
NG = CB // G


def _sc_part(x, maskf):
    mesh = plsc.VectorSubcoreMesh(core_axis_name="c", subcore_axis_name="s")

    @functools.partial(
        pl.kernel,
        mesh=mesh,
        out_type=jax.ShapeDtypeStruct((SC_ROWS, N), jnp.float32),
        compiler_params=pltpu.CompilerParams(needs_layout_passes=False),
        scratch_types=[
            pltpu.VMEM((2, 1, CW), jnp.float32),
            pltpu.VMEM((2, 1, CW), jnp.float32),
            pltpu.VMEM((2, 1, CW), jnp.float32),
            pltpu.SemaphoreType.DMA,
            pltpu.SemaphoreType.DMA,
            pltpu.SemaphoreType.DMA,
            pltpu.SemaphoreType.DMA,
        ],
    )
    def k(x_hbm, m_hbm, out_hbm, xw, mw, ow, sin0, sin1, sout0, sout1):
        wid = lax.axis_index("s") * NC + lax.axis_index("c")
        row = B - SC_ROWS + wid
        orow = wid
        sin = (sin0, sin1)
        sout = (sout0, sout1)

        def start_in(w):
            b = w % 2
            hx = pltpu.async_copy(
                x_hbm.at[pl.ds(row, 1), pl.ds(w * CW, CW)], xw.at[b], sin[b])
            hm = pltpu.async_copy(
                m_hbm.at[pl.ds(orow, 1), pl.ds(w * CW, CW)], mw.at[b], sin[b])
            return (hx, hm)

        pending_in = {0: start_in(0)}
        pending_out = {}
        carry = jnp.float32(0.0)
        for w in range(NWIN):
            b = w % 2
            for h in pending_in.pop(w):
                h.wait()
            if w + 1 < NWIN:
                pending_in[w + 1] = start_in(w + 1)
            if w - 2 in pending_out:
                pending_out.pop(w - 2).wait()

            def body(i, c, b=b):
                off = i * L
                v = xw[b, 0, pl.ds(off, L)] * mw[b, 0, pl.ds(off, L)]
                s = jnp.cumsum(v) + c
                ow[b, 0, pl.ds(off, L)] = s
                return s[L - 1]

            carry = lax.fori_loop(0, WCHUNKS, body, carry)
            pending_out[w] = pltpu.async_copy(
                ow.at[b],
                out_hbm.at[pl.ds(orow, 1), pl.ds(w * CW, CW)], sout[b])
        for w in sorted(pending_out):
            pending_out.pop(w).wait()

    return k(x, maskf)


def _tc_body(x_ref, m_ref, u_ref, o_ref, carry_ref):
    i = pl.program_id(0)

    @pl.when(i == 0)
    def _():
        carry_ref[...] = jnp.zeros_like(carry_ref)

    masked = x_ref[...] * m_ref[...].astype(jnp.float32)
    u = u_ref[...]
    off = carry_ref[...]
    for g in range(NG):
        s = jnp.dot(masked[:, g * G:(g + 1) * G], u,
                    preferred_element_type=jnp.float32)
        o_ref[:, g * G:(g + 1) * G] = s + off
        off = off + jnp.broadcast_to(s[:, G - 1:G], (TC_ROWS, G))
    carry_ref[...] = off


def _tc_part(x, mask):
    u = jnp.triu(jnp.ones((G, G), jnp.float32))
    return pl.pallas_call(
        _tc_body,
        grid=(NBLK,),
        in_specs=[
            pl.BlockSpec((TC_ROWS, CB), lambda i: (0, i)),
            pl.BlockSpec((TC_ROWS, CB), lambda i: (0, i)),
            pl.BlockSpec((G, G), lambda i: (0, 0)),
        ],
        out_specs=pl.BlockSpec((TC_ROWS, CB), lambda i: (0, i)),
        out_shape=jax.ShapeDtypeStruct((TC_ROWS, N), jnp.float32),
        scratch_shapes=[pltpu.VMEM((TC_ROWS, G), jnp.float32)],
    )(x, mask, u)


def kernel(x, mask):
    maskf_sc = mask[TC_ROWS:].astype(jnp.float32)
    sc_out = _sc_part(x, maskf_sc)
    tc_out = _tc_part(x, mask)
    return jnp.concatenate([tc_out, sc_out], axis=0)


# speedup vs baseline: 2.8760x; 2.8760x over previous
import jax
import jax.numpy as jnp
from jax.experimental import pallas as pl
from jax.experimental.pallas import tpu as pltpu

B, N = 128, 8192
CB = 2048
NBLK = N // CB
G = 128
NG = CB // G


def _tc_body(x_ref, m_ref, u_ref, o_ref, carry_ref):
    i = pl.program_id(0)

    @pl.when(i == 0)
    def _():
        carry_ref[...] = jnp.zeros_like(carry_ref)

    masked = x_ref[...] * m_ref[...].astype(jnp.float32)
    u = u_ref[...]
    off = carry_ref[...]
    for g in range(NG):
        s = jnp.dot(masked[:, g * G:(g + 1) * G], u,
                    preferred_element_type=jnp.float32)
        o_ref[:, g * G:(g + 1) * G] = s + off
        off = off + jnp.broadcast_to(s[:, G - 1:G], (B, G))
    carry_ref[...] = off


def kernel(x, mask):
    u = jnp.triu(jnp.ones((G, G), jnp.float32))
    return pl.pallas_call(
        _tc_body,
        grid=(NBLK,),
        in_specs=[
            pl.BlockSpec((B, CB), lambda i: (0, i)),
            pl.BlockSpec((B, CB), lambda i: (0, i)),
            pl.BlockSpec((G, G), lambda i: (0, 0)),
        ],
        out_specs=pl.BlockSpec((B, CB), lambda i: (0, i)),
        out_shape=jax.ShapeDtypeStruct((B, N), jnp.float32),
        scratch_shapes=[pltpu.VMEM((B, G), jnp.float32)],
    )(x, mask, u)
